# Initial kernel scaffold; baseline (speedup 1.0000x reference)
import jax
import jax.numpy as jnp
from jax.experimental import pallas as pl


def kernel(z, embedding):
    raise NotImplementedError("write your pallas kernel here")


# speedup vs baseline: 1.2150x; 1.2150x over previous
import functools

import jax
import jax.numpy as jnp
from jax import lax
from jax.experimental import pallas as pl
from jax.experimental.pallas import tpu as pltpu
from jax.experimental.pallas import tpu_sc as plsc

NUM_CODES = 8192
CODE_DIM = 256
B = 16384
EPS = 1e-08
COMMITMENT_COST = 0.3
DIVERSITY_WEIGHT = 0.001

TB = 256
N_TB = B // TB
CHUNK = 2048
N_CH = NUM_CODES // CHUNK


def _argmax_body(z_ref, emb_ref, idx_ref):
    z = z_ref[...]
    best_v = jnp.full((TB, 1), -jnp.inf, jnp.float32)
    best_i = jnp.zeros((TB, 1), jnp.int32)
    for c in range(N_CH):
        sim = lax.dot_general(
            z, emb_ref[pl.ds(c * CHUNK, CHUNK), :],
            (((1,), (1,)), ((), ())),
            preferred_element_type=jnp.float32,
        )
        v = jnp.max(sim, axis=1, keepdims=True)
        col = lax.broadcasted_iota(jnp.int32, (TB, CHUNK), 1)
        ii = jnp.min(jnp.where(sim == v, col, NUM_CODES), axis=1,
                     keepdims=True) + c * CHUNK
        upd = v > best_v
        best_i = jnp.where(upd, ii, best_i)
        best_v = jnp.where(upd, v, best_v)
    idx_ref[0, 0, :] = best_i[:, 0]


def _argmax_call(z, emb):
    return pl.pallas_call(
        _argmax_body,
        grid=(N_TB,),
        in_specs=[
            pl.BlockSpec((TB, CODE_DIM), lambda i: (i, 0)),
            pl.BlockSpec((NUM_CODES, CODE_DIM), lambda i: (0, 0)),
        ],
        out_specs=pl.BlockSpec((1, 1, TB), lambda i: (i, 0, 0)),
        out_shape=jax.ShapeDtypeStruct((N_TB, 1, TB), jnp.int32),
        compiler_params=pltpu.CompilerParams(
            dimension_semantics=("arbitrary",)),
    )(z, emb)


NC = 2
NS = 16
NW = NC * NS
BPW = B // NW
GCH = 128
NG = BPW // GCH

def _sc_body(emb_hbm, idx_hbm, zq_hbm, cnt_hbm,
             idx2d, rows0, rows1, ones_v, cvmem, cshared, sem0, sem1):
    cid = lax.axis_index("c")
    sid = lax.axis_index("s")
    wid = sid * NC + cid
    base = wid * BPW

    for j in range(NG):
        pltpu.sync_copy(idx_hbm.at[pl.ds(base + j * GCH, GCH)], idx2d.at[j])

    @pl.when(sid == 0)
    def _():
        def zeroloop(k, carry):
            cvmem[pl.ds(k * 16, 16)] = jnp.zeros((16,), jnp.float32)
            return carry
        lax.fori_loop(0, NUM_CODES // 16, zeroloop, 0)
        pltpu.sync_copy(cvmem, cshared)

    for k in range(GCH // 16):
        ones_v[pl.ds(k * 16, 16)] = jnp.ones((16,), jnp.float32)

    plsc.subcore_barrier()

    bufs = (rows0, rows1)
    sems = (sem0, sem1)
    for j in range(NG):
        cp = pltpu.async_copy(emb_hbm.at[idx2d.at[j]], bufs[j % 2],
                              sems[j % 2])
        cp.wait()
        pltpu.sync_copy(bufs[j % 2],
                        zq_hbm.at[pl.ds(base + j * GCH, GCH)])
        pltpu.sync_copy(ones_v, cshared.at[idx2d.at[j]], add=True)

    plsc.subcore_barrier()

    @pl.when(sid == 0)
    def _():
        pltpu.sync_copy(cshared, cvmem)
        pltpu.sync_copy(cvmem, cnt_hbm.at[cid])


@functools.cache
def _sc_gather_fn():
    mesh = plsc.VectorSubcoreMesh(core_axis_name="c", subcore_axis_name="s")
    return pl.kernel(
        _sc_body,
        mesh=mesh,
        out_type=[
            jax.ShapeDtypeStruct((B, CODE_DIM), jnp.float32),
            jax.ShapeDtypeStruct((NC, NUM_CODES), jnp.float32),
        ],
        scratch_types=[
            pltpu.VMEM((NG, GCH), jnp.int32),
            pltpu.VMEM((GCH, CODE_DIM), jnp.float32),
            pltpu.VMEM((GCH, CODE_DIM), jnp.float32),
            pltpu.VMEM((GCH,), jnp.float32),
            pltpu.VMEM((NUM_CODES,), jnp.float32),
            pltpu.VMEM_SHARED((NUM_CODES,), jnp.float32),
            pltpu.SemaphoreType.DMA,
            pltpu.SemaphoreType.DMA,
        ],
    )


RB = 2048
N_RB = B // RB
_MAX_ENT = float(jnp.log(jnp.float32(NUM_CODES)))


def _loss_body(z_ref, zq_ref, cnt_ref, out_ref, acc):
    i = pl.program_id(0)

    @pl.when(i == 0)
    def _():
        acc[0] = 0.0

    d = z_ref[...] - zq_ref[...]
    acc[0] += jnp.sum(d * d)

    @pl.when(i == N_RB - 1)
    def _():
        counts = cnt_ref[0:1, :] + cnt_ref[1:2, :]
        total = jnp.sum(counts)
        probs = counts / total
        ent = -jnp.sum(probs * jnp.log(probs + 1e-10), keepdims=True)
        ent = ent.reshape(1, 1)
        perp = jnp.exp(ent)
        div = (_MAX_ENT - ent) / _MAX_ENT
        mse = jnp.full((1, 1), acc[0] / (B * CODE_DIM), jnp.float32)
        vq = mse * COMMITMENT_COST + mse + DIVERSITY_WEIGHT * div
        col = lax.broadcasted_iota(jnp.int32, (1, 128), 1)
        row = (jnp.where(col == 0, vq, 0.0)
               + jnp.where(col == 1, perp, 0.0)
               + jnp.where(col == 2, div, 0.0))
        out_ref[...] = row


def _loss_call(z, z_q, cnt2):
    return pl.pallas_call(
        _loss_body,
        grid=(N_RB,),
        in_specs=[
            pl.BlockSpec((RB, CODE_DIM), lambda i: (i, 0)),
            pl.BlockSpec((RB, CODE_DIM), lambda i: (i, 0)),
            pl.BlockSpec((NC, NUM_CODES), lambda i: (0, 0)),
        ],
        out_specs=pl.BlockSpec((1, 128), lambda i: (0, 0)),
        out_shape=jax.ShapeDtypeStruct((1, 128), jnp.float32),
        scratch_shapes=[pltpu.SMEM((1,), jnp.float32)],
        compiler_params=pltpu.CompilerParams(
            dimension_semantics=("arbitrary",)),
    )(z, z_q, cnt2)


def kernel(z, embedding):
    z_n = z / (jnp.linalg.norm(z, axis=1, keepdims=True) + EPS)
    emb_n = embedding / (jnp.linalg.norm(embedding, axis=1, keepdims=True) + EPS)
    idx3 = _argmax_call(z_n, emb_n)
    indices = idx3.reshape(B)
    z_q, cnt2 = _sc_gather_fn()(embedding, indices)
    s = _loss_call(z, z_q, cnt2)
    return (z_q, s[0, 0], s[0, 1], indices, s[0, 2])


# speedup vs baseline: 1.9881x; 1.6362x over previous
import functools
import math

import jax
import jax.numpy as jnp
from jax import lax
from jax.experimental import pallas as pl
from jax.experimental.pallas import tpu as pltpu
from jax.experimental.pallas import tpu_sc as plsc

NUM_CODES = 8192
CODE_DIM = 256
B = 16384
EPS = 1e-08
COMMITMENT_COST = 0.3
DIVERSITY_WEIGHT = 0.001

TB = 1024
N_TB = B // TB
CHUNK = 2048
N_CH = NUM_CODES // CHUNK


def _argmax_body(z_ref, emb_ref, idx_ref):
    z = z_ref[...]
    sim = lax.dot_general(
        z, emb_ref[...],
        (((1,), (1,)), ((), ())),
        preferred_element_type=jnp.float32,
    )
    idx_ref[0, 0, :] = jnp.argmax(sim, axis=1).astype(jnp.int32)


def _argmax_call(z, emb):
    return pl.pallas_call(
        _argmax_body,
        grid=(N_TB,),
        in_specs=[
            pl.BlockSpec((TB, CODE_DIM), lambda i: (i, 0)),
            pl.BlockSpec((NUM_CODES, CODE_DIM), lambda i: (0, 0)),
        ],
        out_specs=pl.BlockSpec((1, 1, TB), lambda i: (i, 0, 0)),
        out_shape=jax.ShapeDtypeStruct((N_TB, 1, TB), jnp.int32),
        compiler_params=pltpu.CompilerParams(
            dimension_semantics=("arbitrary",)),
    )(z, emb)


NC = 2
NS = 16
NW = NC * NS
BPW = B // NW
GCH = 128
NG = BPW // GCH

def _sc_body(emb_hbm, idx_hbm, zq_hbm, cnt_hbm,
             idx2d, rows0, rows1, ones_v, cvmem, cshared, sem0, sem1):
    cid = lax.axis_index("c")
    sid = lax.axis_index("s")
    wid = sid * NC + cid
    base = wid * BPW

    for j in range(NG):
        pltpu.sync_copy(idx_hbm.at[pl.ds(base + j * GCH, GCH)], idx2d.at[j])

    @pl.when(sid == 0)
    def _():
        def zeroloop(k, carry):
            cvmem[pl.ds(k * 16, 16)] = jnp.zeros((16,), jnp.float32)
            return carry
        lax.fori_loop(0, NUM_CODES // 16, zeroloop, 0)
        pltpu.sync_copy(cvmem, cshared)

    for k in range(GCH // 16):
        ones_v[pl.ds(k * 16, 16)] = jnp.ones((16,), jnp.float32)

    plsc.subcore_barrier()

    bufs = (rows0, rows1)
    sems = (sem0, sem1)
    for j in range(NG):
        cp = pltpu.async_copy(emb_hbm.at[idx2d.at[j]], bufs[j % 2],
                              sems[j % 2])
        cp.wait()
        pltpu.sync_copy(bufs[j % 2],
                        zq_hbm.at[pl.ds(base + j * GCH, GCH)])
        pltpu.sync_copy(ones_v, cshared.at[idx2d.at[j]], add=True)

    plsc.subcore_barrier()

    @pl.when(sid == 0)
    def _():
        pltpu.sync_copy(cshared, cvmem)
        pltpu.sync_copy(cvmem, cnt_hbm.at[cid])


@functools.cache
def _sc_gather_fn():
    mesh = plsc.VectorSubcoreMesh(core_axis_name="c", subcore_axis_name="s")
    return pl.kernel(
        _sc_body,
        mesh=mesh,
        out_type=[
            jax.ShapeDtypeStruct((B, CODE_DIM), jnp.float32),
            jax.ShapeDtypeStruct((NC, NUM_CODES), jnp.float32),
        ],
        scratch_types=[
            pltpu.VMEM((NG, GCH), jnp.int32),
            pltpu.VMEM((GCH, CODE_DIM), jnp.float32),
            pltpu.VMEM((GCH, CODE_DIM), jnp.float32),
            pltpu.VMEM((GCH,), jnp.float32),
            pltpu.VMEM((NUM_CODES,), jnp.float32),
            pltpu.VMEM_SHARED((NUM_CODES,), jnp.float32),
            pltpu.SemaphoreType.DMA,
            pltpu.SemaphoreType.DMA,
        ],
    )


RB = 2048
N_RB = B // RB
_MAX_ENT = math.log(float(NUM_CODES))


def _loss_body(z_ref, zq_ref, cnt_ref, out_ref, acc):
    i = pl.program_id(0)

    @pl.when(i == 0)
    def _():
        acc[0] = 0.0

    d = z_ref[...] - zq_ref[...]
    acc[0] += jnp.sum(d * d)

    @pl.when(i == N_RB - 1)
    def _():
        counts = cnt_ref[0:1, :] + cnt_ref[1:2, :]
        total = jnp.sum(counts)
        probs = counts / total
        ent = -jnp.sum(probs * jnp.log(probs + 1e-10), keepdims=True)
        ent = ent.reshape(1, 1)
        perp = jnp.exp(ent)
        div = (_MAX_ENT - ent) / _MAX_ENT
        mse = jnp.full((1, 1), acc[0] / (B * CODE_DIM), jnp.float32)
        vq = mse * COMMITMENT_COST + mse + DIVERSITY_WEIGHT * div
        col = lax.broadcasted_iota(jnp.int32, (1, 128), 1)
        row = (jnp.where(col == 0, vq, 0.0)
               + jnp.where(col == 1, perp, 0.0)
               + jnp.where(col == 2, div, 0.0))
        out_ref[...] = row


def _loss_call(z, z_q, cnt2):
    return pl.pallas_call(
        _loss_body,
        grid=(N_RB,),
        in_specs=[
            pl.BlockSpec((RB, CODE_DIM), lambda i: (i, 0)),
            pl.BlockSpec((RB, CODE_DIM), lambda i: (i, 0)),
            pl.BlockSpec((NC, NUM_CODES), lambda i: (0, 0)),
        ],
        out_specs=pl.BlockSpec((1, 128), lambda i: (0, 0)),
        out_shape=jax.ShapeDtypeStruct((1, 128), jnp.float32),
        scratch_shapes=[pltpu.SMEM((1,), jnp.float32)],
        compiler_params=pltpu.CompilerParams(
            dimension_semantics=("arbitrary",)),
    )(z, z_q, cnt2)


def kernel(z, embedding):
    z_n = z / (jnp.linalg.norm(z, axis=1, keepdims=True) + EPS)
    emb_n = embedding / (jnp.linalg.norm(embedding, axis=1, keepdims=True) + EPS)
    idx3 = _argmax_call(z_n.astype(jnp.bfloat16), emb_n.astype(jnp.bfloat16))
    indices = idx3.reshape(B)
    z_q, cnt2 = _sc_gather_fn()(embedding, indices)
    s = _loss_call(z, z_q, cnt2)
    return (z_q, s[0, 0], s[0, 1], indices, s[0, 2])


# speedup vs baseline: 2.0036x; 1.0078x over previous
import functools
import math

import jax
import jax.numpy as jnp
from jax import lax
from jax.experimental import pallas as pl
from jax.experimental.pallas import tpu as pltpu
from jax.experimental.pallas import tpu_sc as plsc

NUM_CODES = 8192
CODE_DIM = 256
B = 16384
EPS = 1e-08
COMMITMENT_COST = 0.3
DIVERSITY_WEIGHT = 0.001

TB = 1024
N_TB = B // TB
CHUNK = 2048
N_CH = NUM_CODES // CHUNK


def _argmax_body(z_ref, emb_ref, idx_ref):
    z = z_ref[...]
    sim = lax.dot_general(
        z, emb_ref[...],
        (((1,), (1,)), ((), ())),
        preferred_element_type=jnp.float32,
    )
    idx_ref[0, 0, :] = jnp.argmax(sim, axis=1).astype(jnp.int32)


def _argmax_call(z, emb):
    return pl.pallas_call(
        _argmax_body,
        grid=(N_TB,),
        in_specs=[
            pl.BlockSpec((TB, CODE_DIM), lambda i: (i, 0)),
            pl.BlockSpec((NUM_CODES, CODE_DIM), lambda i: (0, 0)),
        ],
        out_specs=pl.BlockSpec((1, 1, TB), lambda i: (i, 0, 0)),
        out_shape=jax.ShapeDtypeStruct((N_TB, 1, TB), jnp.int32),
        compiler_params=pltpu.CompilerParams(
            dimension_semantics=("arbitrary",)),
    )(z, emb)


NC = 2
NS = 16
NW = NC * NS
BPW = B // NW
GCH = 128
NG = BPW // GCH

def _sc_body(emb_hbm, idx_hbm, zq_hbm, cnt_hbm,
             idx2d, rows0, rows1, ones_v, cvmem, cshared,
             sem0, sem1, sem2, sem3):
    cid = lax.axis_index("c")
    sid = lax.axis_index("s")
    wid = sid * NC + cid
    base = wid * BPW

    for j in range(NG):
        pltpu.sync_copy(idx_hbm.at[pl.ds(base + j * GCH, GCH)], idx2d.at[j])

    @pl.when(sid == 0)
    def _():
        def zeroloop(k, carry):
            cvmem[pl.ds(k * 16, 16)] = jnp.zeros((16,), jnp.float32)
            return carry
        lax.fori_loop(0, NUM_CODES // 16, zeroloop, 0)
        pltpu.sync_copy(cvmem, cshared)

    for k in range(GCH // 16):
        ones_v[pl.ds(k * 16, 16)] = jnp.ones((16,), jnp.float32)

    plsc.subcore_barrier()

    bufs = (rows0, rows1)
    gsems = (sem0, sem1)
    osems = (sem2, sem3)
    out_cp = [None, None]
    cp_cur = pltpu.async_copy(emb_hbm.at[idx2d.at[0]], bufs[0], gsems[0])
    for j in range(NG):
        nxt = (j + 1) % 2
        if j + 1 < NG:
            if out_cp[nxt] is not None:
                out_cp[nxt].wait()
                out_cp[nxt] = None
            cp_nxt = pltpu.async_copy(emb_hbm.at[idx2d.at[j + 1]],
                                      bufs[nxt], gsems[nxt])
        cp_cur.wait()
        out_cp[j % 2] = pltpu.async_copy(
            bufs[j % 2], zq_hbm.at[pl.ds(base + j * GCH, GCH)], osems[j % 2])
        pltpu.sync_copy(ones_v, cshared.at[idx2d.at[j]], add=True)
        if j + 1 < NG:
            cp_cur = cp_nxt
    for b in range(2):
        if out_cp[b] is not None:
            out_cp[b].wait()

    plsc.subcore_barrier()

    @pl.when(sid == 0)
    def _():
        pltpu.sync_copy(cshared, cvmem)
        pltpu.sync_copy(cvmem, cnt_hbm.at[cid])


@functools.cache
def _sc_gather_fn():
    mesh = plsc.VectorSubcoreMesh(core_axis_name="c", subcore_axis_name="s")
    return pl.kernel(
        _sc_body,
        mesh=mesh,
        out_type=[
            jax.ShapeDtypeStruct((B, CODE_DIM), jnp.float32),
            jax.ShapeDtypeStruct((NC, NUM_CODES), jnp.float32),
        ],
        scratch_types=[
            pltpu.VMEM((NG, GCH), jnp.int32),
            pltpu.VMEM((GCH, CODE_DIM), jnp.float32),
            pltpu.VMEM((GCH, CODE_DIM), jnp.float32),
            pltpu.VMEM((GCH,), jnp.float32),
            pltpu.VMEM((NUM_CODES,), jnp.float32),
            pltpu.VMEM_SHARED((NUM_CODES,), jnp.float32),
            pltpu.SemaphoreType.DMA,
            pltpu.SemaphoreType.DMA,
            pltpu.SemaphoreType.DMA,
            pltpu.SemaphoreType.DMA,
        ],
    )


RB = 2048
N_RB = B // RB
_MAX_ENT = math.log(float(NUM_CODES))


def _loss_body(z_ref, zq_ref, cnt_ref, out_ref, acc):
    i = pl.program_id(0)

    @pl.when(i == 0)
    def _():
        acc[0] = 0.0

    d = z_ref[...] - zq_ref[...]
    acc[0] += jnp.sum(d * d)

    @pl.when(i == N_RB - 1)
    def _():
        counts = cnt_ref[0:1, :] + cnt_ref[1:2, :]
        total = jnp.sum(counts)
        probs = counts / total
        ent = -jnp.sum(probs * jnp.log(probs + 1e-10), axis=1, keepdims=True)
        perp = jnp.exp(ent)
        div = (_MAX_ENT - ent) / _MAX_ENT
        mse = jnp.full((1, 1), acc[0] / (B * CODE_DIM), jnp.float32)
        vq = mse * COMMITMENT_COST + mse + DIVERSITY_WEIGHT * div
        col = lax.broadcasted_iota(jnp.int32, (1, 128), 1)
        row = (jnp.where(col == 0, vq, 0.0)
               + jnp.where(col == 1, perp, 0.0)
               + jnp.where(col == 2, div, 0.0))
        out_ref[...] = row


def _loss_call(z, z_q, cnt2):
    return pl.pallas_call(
        _loss_body,
        grid=(N_RB,),
        in_specs=[
            pl.BlockSpec((RB, CODE_DIM), lambda i: (i, 0)),
            pl.BlockSpec((RB, CODE_DIM), lambda i: (i, 0)),
            pl.BlockSpec((NC, NUM_CODES), lambda i: (0, 0)),
        ],
        out_specs=pl.BlockSpec((1, 128), lambda i: (0, 0)),
        out_shape=jax.ShapeDtypeStruct((1, 128), jnp.float32),
        scratch_shapes=[pltpu.SMEM((1,), jnp.float32)],
        compiler_params=pltpu.CompilerParams(
            dimension_semantics=("arbitrary",)),
    )(z, z_q, cnt2)


def kernel(z, embedding):
    nz = jnp.linalg.norm(z, axis=1, keepdims=True)
    z_n = z / (nz + EPS)
    emb_n = embedding / (jnp.linalg.norm(embedding, axis=1, keepdims=True) + EPS)
    z_nb = z_n.astype(jnp.bfloat16)
    idx3 = _argmax_call(z_nb, emb_n.astype(jnp.bfloat16))
    indices = idx3.reshape(B)
    z_q, cnt2 = _sc_gather_fn()(embedding, indices)
    s = _loss_call(z, z_q, cnt2)
    return (z_q, s[0, 0], s[0, 1], indices, s[0, 2])
